# baseline (speedup 1.0000x reference)
import functools

import jax
import jax.numpy as jnp
from jax import lax
from jax.experimental import pallas as pl
from jax.experimental.pallas import tpu as pltpu
from jax.experimental.pallas import tpu_sc as plsc

VOCAB = 1000000
EMB = 64
BATCH = 4096
SEQ = 200

NC = 2
NS = 16
L = 16
NW = NC * NS
B_PER_W = BATCH // NW
HALF = SEQ // 2
INV_S = 1.0 / SEQ


def _seq_encoder_body(table_hbm, idx_hbm, out_hbm, idx_v, rows_v, out_v, sem):
    wid = lax.axis_index("s") * NC + lax.axis_index("c")
    base = wid * B_PER_W

    pltpu.sync_copy(idx_hbm.at[pl.ds(base * 2, 2 * B_PER_W)], idx_v)

    def per_batch(b, _):
        cp0 = pltpu.async_copy(
            table_hbm.at[idx_v.at[2 * b]], rows_v.at[pl.ds(0, HALF)], sem)
        cp1 = pltpu.async_copy(
            table_hbm.at[idx_v.at[2 * b + 1]], rows_v.at[pl.ds(HALF, HALF)], sem)
        cp0.wait()
        cp1.wait()

        def reduce_step(j, acc):
            a0, a1, a2, a3 = acc
            a0 = a0 + rows_v[j, pl.ds(0 * L, L)]
            a1 = a1 + rows_v[j, pl.ds(1 * L, L)]
            a2 = a2 + rows_v[j, pl.ds(2 * L, L)]
            a3 = a3 + rows_v[j, pl.ds(3 * L, L)]
            return (a0, a1, a2, a3)

        zero = jnp.zeros((L,), jnp.float32)
        a0, a1, a2, a3 = lax.fori_loop(
            0, SEQ, reduce_step, (zero, zero, zero, zero), unroll=4)
        out_v[b, pl.ds(0 * L, L)] = a0 * INV_S
        out_v[b, pl.ds(1 * L, L)] = a1 * INV_S
        out_v[b, pl.ds(2 * L, L)] = a2 * INV_S
        out_v[b, pl.ds(3 * L, L)] = a3 * INV_S
        return _

    lax.fori_loop(0, B_PER_W, per_batch, 0)

    pltpu.sync_copy(out_v, out_hbm.at[pl.ds(base, B_PER_W)])


@functools.partial(jax.jit, static_argnames=())
def _seq_encoder(idx2d, table):
    mesh = plsc.VectorSubcoreMesh(core_axis_name="c", subcore_axis_name="s")
    k = pl.kernel(
        _seq_encoder_body,
        out_type=jax.ShapeDtypeStruct((BATCH, EMB), jnp.float32),
        mesh=mesh,
        compiler_params=pltpu.CompilerParams(use_tc_tiling_on_sc=False),
        scratch_types=[
            pltpu.VMEM((2 * B_PER_W, HALF), jnp.int32),
            pltpu.VMEM((SEQ, EMB), jnp.float32),
            pltpu.VMEM((B_PER_W, EMB), jnp.float32),
            pltpu.SemaphoreType.DMA,
        ],
    )
    return k(table, idx2d)


def kernel(inputs, table):
    idx2d = inputs.astype(jnp.int32).reshape(BATCH * 2, HALF)
    return _seq_encoder(idx2d, table)
